# Initial kernel scaffold; baseline (speedup 1.0000x reference)
import jax
import jax.numpy as jnp
from jax.experimental import pallas as pl


def kernel(feat_flat, W1, b1, W2, b2):
    raise NotImplementedError("write your pallas kernel here")


# speedup vs baseline: 6.6596x; 6.6596x over previous
import functools

import numpy as np
import jax
import jax.numpy as jnp
from jax import lax
from jax.experimental import pallas as pl
from jax.experimental.pallas import tpu as pltpu
from jax.experimental.pallas import tpu_sc as plsc

H_FFT = 384
W_FFT = 193
NUM_BANDS = 6
HIDDEN = 128
F = H_FFT * W_FFT
ROWS = 4 * 96
NC, NS = 2, 16
NW = NC * NS
ROWS_PER_W = ROWS // NW
PIECES = 8
PW = F // PIECES
CHUNKS = PW // 16


def _band_tables():
    yy = np.arange(H_FFT, dtype=np.float32).reshape(-1, 1)
    xx = np.arange(W_FFT, dtype=np.float32).reshape(1, -1)
    ry = yy / max(H_FFT - 1, 1)
    rx = xx / max(W_FFT - 1, 1)
    r = np.sqrt(ry ** 2 + rx ** 2)
    r = r / (r.max() + 1e-8)
    band = np.minimum(np.floor(r * NUM_BANDS), NUM_BANDS - 1)
    band = band.astype(np.int32).reshape(-1)
    counts = np.zeros(NUM_BANDS, dtype=np.float32)
    for b in range(NUM_BANDS):
        counts[b] = max(float((band == b).sum()), 1.0)
    inv = np.float32(1.0) / (counts + np.float32(1e-6))
    return band, [float(v) for v in inv]


_BAND_NP, _INV_COUNTS = _band_tables()

_MESH = plsc.VectorSubcoreMesh(core_axis_name="c", subcore_axis_name="s")


@functools.partial(
    pl.kernel,
    out_type=jax.ShapeDtypeStruct((ROWS, F), jnp.float32),
    mesh=_MESH,
    compiler_params=pltpu.CompilerParams(
        use_tc_tiling_on_sc=False, needs_layout_passes=False),
    scratch_types=[
        pltpu.VMEM((F,), jnp.int32),
        pltpu.VMEM((PW,), jnp.float32),
        pltpu.VMEM((PW,), jnp.float32),
        pltpu.VMEM((NUM_BANDS * HIDDEN,), jnp.float32),
        pltpu.VMEM((HIDDEN,), jnp.float32),
        pltpu.VMEM((HIDDEN * 16,), jnp.float32),
        pltpu.VMEM((16,), jnp.float32),
        pltpu.VMEM((16,), jnp.float32),
    ],
)
def _rbg(feat_hbm, band_hbm, w1_hbm, b1_hbm, w2_hbm, b2_hbm, out_hbm,
         band_v, fbuf, obuf, w1v, b1v, w2v, b2v, alpha_v):
    wid = lax.axis_index("s") * NC + lax.axis_index("c")

    pltpu.sync_copy(band_hbm, band_v)
    pltpu.sync_copy(w1_hbm, w1v)
    pltpu.sync_copy(b1_hbm, b1v)
    pltpu.sync_copy(w2_hbm, w2v)
    pltpu.sync_copy(b2_hbm, b2v)

    zero16 = jnp.zeros((16,), jnp.float32)

    def row_body(r, carry):
        row = wid * ROWS_PER_W + r

        accs = tuple(zero16 for _ in range(NUM_BANDS))
        for p in range(PIECES):
            pltpu.sync_copy(feat_hbm.at[row, pl.ds(p * PW, PW)], fbuf)

            def red_body(i, a, _p=p):
                off = i * 16
                fv = fbuf[pl.ds(off, 16)]
                bv = band_v[pl.ds(_p * PW + off, 16)]
                return tuple(
                    a[k] + jnp.where(bv == k, fv, zero16)
                    for k in range(NUM_BANDS)
                )

            accs = lax.fori_loop(0, CHUNKS, red_body, accs)

        def hsum(v):
            s = v[0]
            for l in range(1, 16):
                s = s + v[l]
            return s

        means = [hsum(accs[k]) * _INV_COUNTS[k] for k in range(NUM_BANDS)]

        h_chunks = []
        for c8 in range(HIDDEN // 16):
            hv = b1v[pl.ds(c8 * 16, 16)]
            for k in range(NUM_BANDS):
                hv = hv + means[k] * w1v[pl.ds(k * HIDDEN + c8 * 16, 16)]
            h_chunks.append(jnp.maximum(hv, 0.0))

        av = b2v[...]
        for c8 in range(HIDDEN // 16):
            for l in range(16):
                j = c8 * 16 + l
                av = av + h_chunks[c8][l] * w2v[pl.ds(j * 16, 16)]
        alpha_v[...] = 1.0 / (1.0 + jnp.exp(-av))

        for p in range(PIECES):

            def exp_body(i, c, _p=p):
                off = i * 16
                bv = band_v[pl.ds(_p * PW + off, 16)]
                obuf[pl.ds(off, 16)] = plsc.load_gather(alpha_v, [bv])
                return c

            lax.fori_loop(0, CHUNKS, exp_body, 0)
            pltpu.sync_copy(obuf, out_hbm.at[row, pl.ds(p * PW, PW)])
        return carry

    lax.fori_loop(0, ROWS_PER_W, row_body, 0)


def kernel(feat_flat, W1, b1, W2, b2):
    B, C, Fdim = feat_flat.shape
    feat2 = feat_flat.reshape(B * C, Fdim)
    w2p = jnp.zeros((HIDDEN, 16), W2.dtype).at[:, :NUM_BANDS].set(W2)
    b2p = jnp.zeros((16,), b2.dtype).at[:NUM_BANDS].set(b2)
    out = _rbg(feat2, jnp.asarray(_BAND_NP), W1.reshape(-1), b1,
               w2p.reshape(-1), b2p)
    return out.reshape(B, C, Fdim)
